# baseline (speedup 1.0000x reference)
import functools

import jax
import jax.numpy as jnp
from jax import lax
from jax.experimental import pallas as pl
from jax.experimental.pallas import tpu as pltpu
from jax.experimental.pallas import tpu_sc as plsc

DIM = 1024
HID = 1024
E = 16
T = 16
LIMIT = 7.0
EPS = 1e-5

NC1 = 4
NC2 = 2
R1 = HID // NC1
R2 = DIM // NC2

L = 16
NCH = DIM // L


def _router_sc(x_hbm, nw_hbm, gw_hbm, gb_hbm, xn_hbm, wd_hbm,
               xv, nwv, gwv, gbv, xnv, wdv):
    t = lax.axis_index("s")
    iota = lax.iota(jnp.int32, L)

    dnums = lax.GatherDimensionNumbers(offset_dims=(),
                                       collapsed_slice_dims=(0,),
                                       start_index_map=(0,))

    def _shuffle(vec, idx):
        return lax.gather(vec, idx[:, None], dnums, (1,),
                          mode=lax.GatherScatterMode.PROMISE_IN_BOUNDS)

    def _butterfly(vec, op):
        for sh in (1, 2, 4, 8):
            vec = op(vec, _shuffle(vec, iota ^ sh))
        return vec

    def hsum(vec):
        return _butterfly(vec, jnp.add)

    def hmax(vec):
        return _butterfly(vec, jnp.maximum)

    @pl.when(lax.axis_index("c") == 0)
    def _work():
        pltpu.sync_copy(x_hbm.at[t], xv)
        pltpu.sync_copy(nw_hbm, nwv)
        pltpu.sync_copy(gw_hbm, gwv)
        pltpu.sync_copy(gb_hbm, gbv)

        acc = jnp.zeros((L,), jnp.float32)
        for k in range(NCH):
            c = xv[k * L:(k + 1) * L]
            acc = acc + c * c
        v = hsum(acc) * (1.0 / DIM) + EPS
        iv = plsc.bitcast(v, jnp.int32)
        y = plsc.bitcast(jnp.full((L,), 0x5F3759DF, jnp.int32) - (iv >> 1),
                         jnp.float32)
        for _ in range(3):
            y = y * (1.5 - 0.5 * v * y * y)
        s = y

        for k in range(NCH):
            xnv[k * L:(k + 1) * L] = (xv[k * L:(k + 1) * L]
                                      * nwv[k * L:(k + 1) * L] * s)
        pltpu.sync_copy(xnv, xn_hbm.at[t])

        g = gbv[...]
        for e in range(E):
            d = jnp.zeros((L,), jnp.float32)
            for k in range(NCH):
                d = d + (xnv[k * L:(k + 1) * L]
                         * gwv[e * DIM + k * L:e * DIM + (k + 1) * L])
            g = g + jnp.where(iota == e, hsum(d), 0.0)

        m1 = hmax(g)
        idx1 = plsc.all_reduce_ffs(g == m1)
        mask1 = iota == idx1
        g2 = jnp.where(mask1, -jnp.inf, g)
        m2 = hmax(g2)
        idx2 = plsc.all_reduce_ffs(g2 == m2)
        mask2 = iota == idx2
        e2 = jnp.exp(m2 - m1)
        den = 1.0 + e2
        wdv[...] = (jnp.where(mask1, 1.0 / den, 0.0)
                    + jnp.where(mask2, e2 / den, 0.0))
        pltpu.sync_copy(wdv, wd_hbm.at[t])


def _moe(xn_in, wd_in, x_ref, b1g_ref, b1l_ref, b2_ref,
         w1_hbm, w2_hbm, out_ref, w1_buf, w2_buf, sem1, sem2):
    i = pl.program_id(0)

    def start(src, buf):
        for c in range(NC1):
            pltpu.make_async_copy(
                w1_hbm.at[src, pl.ds(c * R1, R1), :],
                w1_buf.at[buf, pl.ds(c * R1, R1), :],
                sem1.at[buf, c]).start()
        for c in range(NC2):
            pltpu.make_async_copy(
                w2_hbm.at[src, pl.ds(c * R2, R2), :],
                w2_buf.at[buf, pl.ds(c * R2, R2), :],
                sem2.at[buf, c]).start()

    def wait(buf):
        for c in range(NC1):
            pltpu.make_async_copy(
                w1_hbm.at[0, pl.ds(c * R1, R1), :],
                w1_buf.at[buf, pl.ds(c * R1, R1), :],
                sem1.at[buf, c]).wait()
        for c in range(NC2):
            pltpu.make_async_copy(
                w2_hbm.at[0, pl.ds(c * R2, R2), :],
                w2_buf.at[buf, pl.ds(c * R2, R2), :],
                sem2.at[buf, c]).wait()

    slot = jax.lax.rem(i, 2)
    nxt = jax.lax.rem(i + 1, 2)

    @pl.when(i == 0)
    def _prologue():
        start(0, 0)
        out_ref[...] = x_ref[...]

    @pl.when(i + 1 < E)
    def _prefetch():
        start(i + 1, nxt)

    wait(slot)
    xnb = xn_in[...].astype(jnp.bfloat16)
    w1 = w1_buf[slot].astype(jnp.bfloat16)
    b1g = b1g_ref[pl.ds(i, 1), :]
    b1l = b1l_ref[pl.ds(i, 1), :]
    hg = jax.lax.dot_general(xnb, w1[:, :DIM], (((1,), (1,)), ((), ())),
                             preferred_element_type=jnp.float32) + b1g
    hl = jax.lax.dot_general(xnb, w1[:, DIM:], (((1,), (1,)), ((), ())),
                             preferred_element_type=jnp.float32) + b1l
    hg = jnp.minimum(hg, LIMIT)
    hl = jnp.clip(hl, -LIMIT, LIMIT)
    act = hg * jax.nn.sigmoid(1.702 * hg) * (hl + 1.0)
    actb = act.astype(jnp.bfloat16)
    w2b = w2_buf[slot].astype(jnp.bfloat16)
    y = jax.lax.dot_general(actb, w2b, (((1,), (1,)), ((), ())),
                            preferred_element_type=jnp.float32)
    y = y + b2_ref[pl.ds(i, 1), :]
    iota = jax.lax.broadcasted_iota(jnp.int32, (T, E), 1)
    wcol = jnp.sum(jnp.where(iota == i, wd_in[...], 0.0), axis=1,
                   keepdims=True)
    out_ref[...] += wcol * y


def kernel(x, freqs_cos, freqs_sin, gate_w, gate_b, w1, b1, w2, b2, norm_w):
    del freqs_cos, freqs_sin
    w1r = w1.reshape(E, HID, 2 * DIM)
    b1g = b1[:, 0::2]
    b1l = b1[:, 1::2]

    router = functools.partial(
        pl.kernel,
        mesh=plsc.VectorSubcoreMesh(core_axis_name="c", subcore_axis_name="s"),
        out_type=[
            jax.ShapeDtypeStruct((T, DIM), jnp.float32),
            jax.ShapeDtypeStruct((T, E), jnp.float32),
        ],
        scratch_types=[
            pltpu.VMEM((DIM,), jnp.float32),
            pltpu.VMEM((DIM,), jnp.float32),
            pltpu.VMEM((E * DIM,), jnp.float32),
            pltpu.VMEM((E,), jnp.float32),
            pltpu.VMEM((DIM,), jnp.float32),
            pltpu.VMEM((E,), jnp.float32),
        ],
        compiler_params=pltpu.CompilerParams(needs_layout_passes=False),
    )(_router_sc)
    xn, wd = router(x, norm_w, gate_w.reshape(E * DIM), gate_b)

    full = lambda shape: pl.BlockSpec(shape, lambda i: (0,) * len(shape))
    hbm = pl.BlockSpec(memory_space=pltpu.MemorySpace.HBM)

    return pl.pallas_call(
        _moe,
        grid=(E,),
        in_specs=[
            full((T, DIM)),
            full((T, E)),
            full((T, DIM)),
            full((E, HID)),
            full((E, HID)),
            full((E, DIM)),
            hbm,
            hbm,
        ],
        out_specs=full((T, DIM)),
        out_shape=jax.ShapeDtypeStruct((T, DIM), jnp.float32),
        scratch_shapes=[
            pltpu.VMEM((2, HID, 2 * DIM), jnp.float32),
            pltpu.VMEM((2, DIM, HID), jnp.float32),
            pltpu.SemaphoreType.DMA((2, NC1)),
            pltpu.SemaphoreType.DMA((2, NC2)),
        ],
        compiler_params=pltpu.CompilerParams(
            dimension_semantics=("arbitrary",),
        ),
    )(xn, wd, x, b1g, b1l, b2, w1r, w2)
